# Initial kernel scaffold; baseline (speedup 1.0000x reference)
import jax
import jax.numpy as jnp
from jax.experimental import pallas as pl


def kernel(x_a, edge_index_a2a, edge_attr_a2a, g_W, g_b, f_W, f_b, cls_W, cls_b):
    raise NotImplementedError("write your pallas kernel here")


# speedup vs baseline: 16.3392x; 16.3392x over previous
import functools

import jax
import jax.numpy as jnp
from jax import lax
from jax.experimental import pallas as pl
from jax.experimental.pallas import tpu as pltpu
from jax.experimental.pallas import tpu_sc as plsc

N_NODES = 10000
N_EDGES = 320000
CH = 128
EA = 16

NC = 2
NS = 16
L = 16
NW = NC * NS
EPW = N_EDGES // NW
STEPS = EPW // L
ZSTEPS = N_NODES // L
E128 = N_EDGES * EA // CH

_HI = lax.Precision.HIGHEST


def _prep_body(x_ref, gW_ref, gb_ref, fW_ref, fb_ref, cW_ref, cb_ref,
               node_ref, P_ref):
    cW = cW_ref[...]
    fW = fW_ref[...]
    u = lax.dot_general(fW[0:CH], cW, (((1,), (0,)), ((), ())), precision=_HI)
    v = lax.dot_general(fW[CH:2 * CH], cW, (((1,), (0,)), ((), ())), precision=_HI)
    gW = gW_ref[...]
    a = lax.dot_general(gW[0:CH], v, (((1,), (0,)), ((), ())), precision=_HI)
    b = lax.dot_general(gW[CH:2 * CH], v, (((1,), (0,)), ((), ())), precision=_HI)
    c = lax.dot_general(gW[2 * CH:2 * CH + EA], v, (((1,), (0,)), ((), ())), precision=_HI)
    s0 = jnp.sum(gb_ref[...] * v[:, 0])
    s1 = jnp.sum(fb_ref[...] * cW[:, 0]) + jnp.sum(cb_ref[...])
    W3 = jnp.concatenate([u, a, b, jnp.zeros((CH, 5), jnp.float32)], axis=1)
    node = lax.dot_general(W3, x_ref[...], (((0,), (1,)), ((), ())), precision=_HI)
    row = lax.broadcasted_iota(jnp.int32, node.shape, 0)
    node_ref[...] = node + jnp.where(row == 0, s1, 0.0) + jnp.where(row == 1, s0, 0.0)
    ctile = jnp.concatenate([c] * 8, axis=0)
    i0 = lax.broadcasted_iota(jnp.int32, (CH, 8), 0)
    i1 = lax.broadcasted_iota(jnp.int32, (CH, 8), 1)
    P_ref[...] = jnp.where(i0 // EA == i1, ctile, 0.0)


def _edge_body(X_ref, P_ref, out_ref):
    out_ref[...] = lax.dot_general(X_ref[...], P_ref[...],
                                   (((1,), (0,)), ((), ())), precision=_HI)


def _combine_body(node_ref, T_ref, cnt_ref, out_ref):
    T = jnp.sum(T_ref[...], axis=0, keepdims=True)
    cnt = jnp.sum(cnt_ref[...], axis=0, keepdims=True)
    xu = node_ref[0:1, :]
    xa = node_ref[1:2, :]
    out_ref[...] = xu + jnp.where(cnt > 0.0, xa + T / jnp.maximum(cnt, 1.0), 0.0)


@functools.cache
def _sc_segsum_kernel():
    return pl.kernel(
        _sc_segsum_body,
        out_type=[jax.ShapeDtypeStruct((NW, N_NODES), jnp.float32),
                  jax.ShapeDtypeStruct((NW, N_NODES), jnp.float32)],
        mesh=plsc.VectorSubcoreMesh(core_axis_name="c", subcore_axis_name="s",
                                    num_cores=NC, num_subcores=NS),
        compiler_params=pltpu.CompilerParams(needs_layout_passes=False),
        scratch_types=[pltpu.VMEM((EPW,), jnp.int32),
                       pltpu.VMEM((EPW,), jnp.int32),
                       pltpu.VMEM((EPW,), jnp.float32),
                       pltpu.VMEM((N_NODES,), jnp.float32),
                       pltpu.VMEM((N_NODES,), jnp.float32),
                       pltpu.VMEM((N_NODES,), jnp.float32)],
    )


def _sc_segsum_body(src_hbm, dst_hbm, r_hbm, p_hbm, T_hbm, cnt_hbm,
                    src_v, dst_v, r_v, p_v, T_v, cnt_v):
    wid = lax.axis_index("s") * NC + lax.axis_index("c")
    base = wid * EPW
    pltpu.sync_copy(src_hbm.at[pl.ds(base, EPW)], src_v)
    pltpu.sync_copy(dst_hbm.at[pl.ds(base, EPW)], dst_v)
    pltpu.sync_copy(r_hbm.at[pl.ds(base, EPW)], r_v)
    pltpu.sync_copy(p_hbm, p_v)

    def zero_body(i, carry):
        off = i * L
        z = jnp.zeros((L,), jnp.float32)
        T_v[pl.ds(off, L)] = z
        cnt_v[pl.ds(off, L)] = z
        return carry

    lax.fori_loop(0, ZSTEPS, zero_body, 0)

    def body(i, carry):
        off = i * L
        s = src_v[pl.ds(off, L)]
        d = dst_v[pl.ds(off, L)]
        rv = r_v[pl.ds(off, L)]
        pv = plsc.load_gather(p_v, [s])
        plsc.addupdate_scatter(T_v, [d], pv + rv)
        plsc.addupdate_scatter(cnt_v, [d], jnp.full((L,), 1.0, jnp.float32))
        return carry

    lax.fori_loop(0, STEPS, body, 0)

    pltpu.sync_copy(T_v, T_hbm.at[wid])
    pltpu.sync_copy(cnt_v, cnt_hbm.at[wid])


def _segment_parts(src, dst, r, p):
    return _sc_segsum_kernel()(src, dst, r, p)


def kernel(x_a, edge_index_a2a, edge_attr_a2a, g_W, g_b, f_W, f_b, cls_W, cls_b):
    src = edge_index_a2a[0]
    dst = edge_index_a2a[1]

    node_out, P = pl.pallas_call(
        _prep_body,
        out_shape=[jax.ShapeDtypeStruct((8, N_NODES), jnp.float32),
                   jax.ShapeDtypeStruct((CH, 8), jnp.float32)],
    )(x_a, g_W, g_b, f_W, f_b, cls_W, cls_b)

    X128 = edge_attr_a2a.reshape(E128, CH)
    r8 = pl.pallas_call(
        _edge_body,
        grid=(10,),
        in_specs=[pl.BlockSpec((E128 // 10, CH), lambda i: (i, 0)),
                  pl.BlockSpec((CH, 8), lambda i: (0, 0))],
        out_specs=pl.BlockSpec((E128 // 10, 8), lambda i: (i, 0)),
        out_shape=jax.ShapeDtypeStruct((E128, 8), jnp.float32),
    )(X128, P)
    r = r8.reshape(-1)
    p = node_out[2]

    T_parts, cnt_parts = _segment_parts(src, dst, r, p)

    out_row = pl.pallas_call(
        _combine_body,
        out_shape=jax.ShapeDtypeStruct((1, N_NODES), jnp.float32),
    )(node_out, T_parts, cnt_parts)
    return out_row.reshape(N_NODES, 1)


# speedup vs baseline: 35.0575x; 2.1456x over previous
import functools

import jax
import jax.numpy as jnp
from jax import lax
from jax.experimental import pallas as pl
from jax.experimental.pallas import tpu as pltpu
from jax.experimental.pallas import tpu_sc as plsc

N_NODES = 10000
N_EDGES = 320000
CH = 128
EA = 16

NC = 2
NS = 16
L = 16
NW = NC * NS
EPW = N_EDGES // NW
STEPS = EPW // L
ZSTEPS = N_NODES // L
E128 = N_EDGES * EA // CH

_HI = lax.Precision.HIGHEST


def _prep_body(x_ref, gW_ref, gb_ref, fW_ref, fb_ref, cW_ref, cb_ref,
               node_ref, c_ref):
    cW = cW_ref[...]
    fW = fW_ref[...]
    u = lax.dot_general(fW[0:CH], cW, (((1,), (0,)), ((), ())), precision=_HI)
    v = lax.dot_general(fW[CH:2 * CH], cW, (((1,), (0,)), ((), ())), precision=_HI)
    gW = gW_ref[...]
    a = lax.dot_general(gW[0:CH], v, (((1,), (0,)), ((), ())), precision=_HI)
    b = lax.dot_general(gW[CH:2 * CH], v, (((1,), (0,)), ((), ())), precision=_HI)
    c = lax.dot_general(gW[2 * CH:2 * CH + EA], v, (((1,), (0,)), ((), ())), precision=_HI)
    s0 = jnp.sum(gb_ref[...] * v[:, 0])
    s1 = jnp.sum(fb_ref[...] * cW[:, 0]) + jnp.sum(cb_ref[...])
    W3 = jnp.concatenate([u, a, b, jnp.zeros((CH, 5), jnp.float32)], axis=1)
    node = lax.dot_general(W3, x_ref[...], (((0,), (1,)), ((), ())), precision=_HI)
    row = lax.broadcasted_iota(jnp.int32, node.shape, 0)
    node_ref[...] = node + jnp.where(row == 0, s1, 0.0) + jnp.where(row == 1, s0, 0.0)
    c_ref[...] = c


def _edge_body(X_ref, c_ref, out_ref):
    j = pl.program_id(0)
    contrib = X_ref[0] * c_ref[pl.ds(j, 1), 0:1]

    @pl.when(j == 0)
    def _():
        out_ref[...] = contrib

    @pl.when(j > 0)
    def _():
        out_ref[...] = out_ref[...] + contrib


def _combine_body(node_ref, T_ref, cnt_ref, out_ref):
    T = jnp.sum(T_ref[...], axis=0, keepdims=True)
    cnt = jnp.sum(cnt_ref[...], axis=0, keepdims=True)
    xu = node_ref[0:1, :]
    xa = node_ref[1:2, :]
    out_ref[...] = xu + jnp.where(cnt > 0.0, xa + T / jnp.maximum(cnt, 1.0), 0.0)


@functools.cache
def _sc_segsum_kernel():
    return pl.kernel(
        _sc_segsum_body,
        out_type=[jax.ShapeDtypeStruct((NW, N_NODES), jnp.float32),
                  jax.ShapeDtypeStruct((NW, N_NODES), jnp.float32)],
        mesh=plsc.VectorSubcoreMesh(core_axis_name="c", subcore_axis_name="s",
                                    num_cores=NC, num_subcores=NS),
        compiler_params=pltpu.CompilerParams(needs_layout_passes=False),
        scratch_types=[pltpu.VMEM((EPW,), jnp.int32),
                       pltpu.VMEM((EPW,), jnp.int32),
                       pltpu.VMEM((EPW,), jnp.float32),
                       pltpu.VMEM((N_NODES,), jnp.float32),
                       pltpu.VMEM((N_NODES,), jnp.float32),
                       pltpu.VMEM((N_NODES,), jnp.float32)],
    )


def _sc_segsum_body(src_hbm, dst_hbm, r_hbm, p_hbm, T_hbm, cnt_hbm,
                    src_v, dst_v, r_v, p_v, T_v, cnt_v):
    wid = lax.axis_index("s") * NC + lax.axis_index("c")
    base = wid * EPW
    pltpu.sync_copy(src_hbm.at[pl.ds(base, EPW)], src_v)
    pltpu.sync_copy(dst_hbm.at[pl.ds(base, EPW)], dst_v)
    pltpu.sync_copy(r_hbm.at[pl.ds(base, EPW)], r_v)
    pltpu.sync_copy(p_hbm, p_v)

    def zero_body(i, carry):
        off = i * L
        z = jnp.zeros((L,), jnp.float32)
        T_v[pl.ds(off, L)] = z
        cnt_v[pl.ds(off, L)] = z
        return carry

    lax.fori_loop(0, ZSTEPS, zero_body, 0)

    def body(i, carry):
        off = i * L
        s = src_v[pl.ds(off, L)]
        d = dst_v[pl.ds(off, L)]
        rv = r_v[pl.ds(off, L)]
        pv = plsc.load_gather(p_v, [s])
        plsc.addupdate_scatter(T_v, [d], pv + rv)
        plsc.addupdate_scatter(cnt_v, [d], jnp.full((L,), 1.0, jnp.float32))
        return carry

    lax.fori_loop(0, STEPS, body, 0)

    pltpu.sync_copy(T_v, T_hbm.at[wid])
    pltpu.sync_copy(cnt_v, cnt_hbm.at[wid])


def _segment_parts(src, dst, r, p):
    return _sc_segsum_kernel()(src, dst, r, p)


def kernel(x_a, edge_index_a2a, edge_attr_a2a, g_W, g_b, f_W, f_b, cls_W, cls_b):
    src = edge_index_a2a[0]
    dst = edge_index_a2a[1]

    node_out, c_vec = pl.pallas_call(
        _prep_body,
        out_shape=[jax.ShapeDtypeStruct((8, N_NODES), jnp.float32),
                   jax.ShapeDtypeStruct((EA, 1), jnp.float32)],
    )(x_a, g_W, g_b, f_W, f_b, cls_W, cls_b)

    QE = N_EDGES // CH
    eaT3 = edge_attr_a2a.T.reshape(EA, QE, CH)
    r2 = pl.pallas_call(
        _edge_body,
        grid=(EA,),
        in_specs=[pl.BlockSpec((1, QE, CH), lambda j: (j, 0, 0)),
                  pl.BlockSpec((EA, 1), lambda j: (0, 0))],
        out_specs=pl.BlockSpec((QE, CH), lambda j: (0, 0)),
        out_shape=jax.ShapeDtypeStruct((QE, CH), jnp.float32),
    )(eaT3, c_vec)
    r = r2.reshape(-1)
    p = node_out[2]

    T_parts, cnt_parts = _segment_parts(src, dst, r, p)

    out_row = pl.pallas_call(
        _combine_body,
        out_shape=jax.ShapeDtypeStruct((1, N_NODES), jnp.float32),
    )(node_out, T_parts, cnt_parts)
    return out_row.reshape(N_NODES, 1)
